# baseline (speedup 1.0000x reference)
import functools

import numpy as np
import jax
import jax.numpy as jnp
from jax import lax
from jax.experimental import pallas as pl
from jax.experimental.pallas import tpu as pltpu
from jax.experimental.pallas import tpu_sc as plsc

_NC = 2
_NS = 16
_NW = _NC * _NS
_L = 16


def _pe_table(seq_len, d_model):
    p = np.arange(seq_len, dtype=np.float64)[:, None]
    i = np.arange(d_model, dtype=np.float64)[None, :]
    i_even = np.where(np.arange(d_model) % 2 == 0, i, i - 1.0)
    angle = p / (10000.0 ** (i_even / d_model))
    pe = np.where(np.arange(d_model) % 2 == 0, np.sin(angle), np.cos(angle))
    return jnp.asarray(pe, dtype=jnp.float32)


@functools.partial(jax.jit, static_argnums=())
def _embed(idx_flat, table, pe):
    rows, = idx_flat.shape
    _, d = table.shape
    seq, _ = pe.shape
    batch = rows // seq

    rows_per_w = rows // _NW
    chunk = 32
    pe_chunk = chunk // batch
    n_steps = rows_per_w // chunk
    dk = d // _L

    mesh = plsc.VectorSubcoreMesh(
        core_axis_name="c", subcore_axis_name="s",
        num_cores=_NC, num_subcores=_NS)

    @functools.partial(
        pl.kernel,
        out_type=jax.ShapeDtypeStruct((rows, d), jnp.float32),
        mesh=mesh,
        scratch_types=[
            pltpu.VMEM((rows_per_w,), jnp.int32),
            pltpu.VMEM((chunk, d), jnp.float32),
            pltpu.VMEM((pe_chunk, d), jnp.float32),
            pltpu.SemaphoreType.DMA,
        ],
    )
    def body(table_hbm, idx_hbm, pe_hbm, out_hbm, idx_v, buf_v, pe_v, sem):
        wid = lax.axis_index("s") * _NC + lax.axis_index("c")
        base = wid * rows_per_w
        pbase = wid * (rows_per_w // batch)
        pltpu.sync_copy(idx_hbm.at[pl.ds(base, rows_per_w)], idx_v)
        for c in range(n_steps):
            gather = pltpu.async_copy(
                table_hbm.at[idx_v.at[pl.ds(c * chunk, chunk)]], buf_v, sem)
            pltpu.sync_copy(
                pe_hbm.at[pl.ds(pbase + c * pe_chunk, pe_chunk)], pe_v)
            gather.wait()

            def add_pe(k, carry):
                for pj in range(pe_chunk):
                    pv = pe_v[pj, pl.ds(k * _L, _L)]
                    for b in range(batch):
                        r = pj * batch + b
                        buf_v[r, pl.ds(k * _L, _L)] = (
                            buf_v[r, pl.ds(k * _L, _L)] + pv)
                return carry

            lax.fori_loop(0, dk, add_pe, 0)
            pltpu.sync_copy(buf_v, out_hbm.at[pl.ds(base + c * chunk, chunk)])

    return body(table, idx_flat, pe)


def kernel(input, table):
    seq, batch = input.shape
    _, d = table.shape
    pe = _pe_table(seq, d)
    idx_flat = input.reshape(seq * batch)
    out = _embed(idx_flat, table, pe)
    return out.reshape(seq, batch, d)


# speedup vs baseline: 1.2497x; 1.2497x over previous
import functools

import numpy as np
import jax
import jax.numpy as jnp
from jax import lax
from jax.experimental import pallas as pl
from jax.experimental.pallas import tpu as pltpu
from jax.experimental.pallas import tpu_sc as plsc

_NC = 2
_NS = 16
_NW = _NC * _NS
_L = 16


def _pe_table(seq_len, d_model):
    p = np.arange(seq_len, dtype=np.float64)[:, None]
    i = np.arange(d_model, dtype=np.float64)[None, :]
    i_even = np.where(np.arange(d_model) % 2 == 0, i, i - 1.0)
    angle = p / (10000.0 ** (i_even / d_model))
    pe = np.where(np.arange(d_model) % 2 == 0, np.sin(angle), np.cos(angle))
    return jnp.asarray(pe, dtype=jnp.float32)


@functools.partial(jax.jit, static_argnums=())
def _embed(idx_flat, table, pe):
    rows, = idx_flat.shape
    _, d = table.shape
    seq, _ = pe.shape
    batch = rows // seq

    rows_per_w = rows // _NW
    chunk = 32
    pe_chunk = chunk // batch
    n_steps = rows_per_w // chunk
    dk = d // _L

    nbuf = 3

    mesh = plsc.VectorSubcoreMesh(
        core_axis_name="c", subcore_axis_name="s",
        num_cores=_NC, num_subcores=_NS)

    @functools.partial(
        pl.kernel,
        out_type=jax.ShapeDtypeStruct((rows, d), jnp.float32),
        mesh=mesh,
        scratch_types=[
            pltpu.VMEM((rows_per_w,), jnp.int32),
            pltpu.VMEM((nbuf, chunk, d), jnp.float32),
            pltpu.VMEM((nbuf, pe_chunk, d), jnp.float32),
            [pltpu.SemaphoreType.DMA] * nbuf,
            [pltpu.SemaphoreType.DMA] * nbuf,
            [pltpu.SemaphoreType.DMA] * nbuf,
        ],
    )
    def body(table_hbm, idx_hbm, pe_hbm, out_hbm, idx_v, buf_v, pe_v,
             gsem, psem, osem):
        wid = lax.axis_index("s") * _NC + lax.axis_index("c")
        base = wid * rows_per_w
        pbase = wid * (rows_per_w // batch)
        pltpu.sync_copy(idx_hbm.at[pl.ds(base, rows_per_w)], idx_v)

        def start_fetch(c):
            s = c % nbuf
            g = pltpu.async_copy(
                table_hbm.at[idx_v.at[pl.ds(c * chunk, chunk)]],
                buf_v.at[s], gsem[s])
            p = pltpu.async_copy(
                pe_hbm.at[pl.ds(pbase + c * pe_chunk, pe_chunk)],
                pe_v.at[s], psem[s])
            return g, p

        fetches = {}
        outs = {}
        for c in range(min(2, n_steps)):
            fetches[c] = start_fetch(c)

        for c in range(n_steps):
            s = c % nbuf
            if c + 2 < n_steps:
                if c >= 1:
                    outs[c - 1].wait()
                fetches[c + 2] = start_fetch(c + 2)
            g, p = fetches.pop(c)
            g.wait()
            p.wait()

            def add_pe(k, carry):
                for pj in range(pe_chunk):
                    pv = pe_v[s, pj, pl.ds(k * _L, _L)]
                    for b in range(batch):
                        r = pj * batch + b
                        buf_v[s, r, pl.ds(k * _L, _L)] = (
                            buf_v[s, r, pl.ds(k * _L, _L)] + pv)
                return carry

            lax.fori_loop(0, dk, add_pe, 0)
            outs[c] = pltpu.async_copy(
                buf_v.at[s], out_hbm.at[pl.ds(base + c * chunk, chunk)],
                osem[s])
        for c in range(max(0, n_steps - nbuf), n_steps):
            if c in outs:
                outs[c].wait()

    return body(table, idx_flat, pe)


def kernel(input, table):
    seq, batch = input.shape
    _, d = table.shape
    pe = _pe_table(seq, d)
    idx_flat = input.reshape(seq * batch)
    out = _embed(idx_flat, table, pe)
    return out.reshape(seq, batch, d)
